# Initial kernel scaffold; baseline (speedup 1.0000x reference)
import jax
import jax.numpy as jnp
from jax.experimental import pallas as pl


def kernel(x_d_user, x_c_user, x_d_item, x_c_item, edge_index_u2i, edge_index_i2u, emb_user, emb_item, W_c_user, b_c_user, W_c_item, b_c_item, Wn_u2i, Wr_u2i, b_u2i, Wn_i2u, Wr_i2u, b_i2u, W_out, b_out):
    raise NotImplementedError("write your pallas kernel here")


# speedup vs baseline: 2.1333x; 2.1333x over previous
import functools

import jax
import jax.numpy as jnp
from jax import lax
from jax.experimental import pallas as pl
from jax.experimental.pallas import tpu as pltpu
from jax.experimental.pallas import tpu_sc as plsc

N = 50000
E = 400000
FD = 4
V = 1000
FC = 16
D = 64
HD = D // 2
OUT = 32

NP = 50176
HALF = NP // 2
SCR = HALF + 32
NCHUNK = NP // 128
ECHUNK = E // 128
ECPT = 196
ROWS_PT = HALF // 16
FB = ROWS_PT // 4

_mesh = functools.partial(
    plsc.VectorSubcoreMesh, core_axis_name="c", subcore_axis_name="s",
    num_cores=2, num_subcores=16)

_SC_PARAMS = pltpu.CompilerParams(use_tc_tiling_on_sc=False)

BLK = 1792


def _f32(*shape):
    return jax.ShapeDtypeStruct(shape, jnp.float32)


def _i32(*shape):
    return jax.ShapeDtypeStruct(shape, jnp.int32)


def _sc_prep_body(idxu, embu, contu, idxi, embi, conti, du_, di_,
                  zeros16, ones16,
                  yu, yi, cnti2, cntu2, dlu, dli,
                  idxv, gbuf, cbuf, ybuf, dbuf, lbuf, onev, fbuf16,
                  sem, cspm_a, cspm_b):
    c = lax.axis_index("c")
    s = lax.axis_index("s")

    emb_off = c * (FD * V)
    y_off = c * NP
    for idxT, embf, cont, yout in ((idxu, embu, contu, yu),
                                   (idxi, embi, conti, yi)):
        nk = (NCHUNK - s + 15) // 16

        def ebody(k, _, idxT=idxT, embf=embf, cont=cont, yout=yout):
            base = (s + 16 * k) * 128
            for j in range(FD):
                pltpu.sync_copy(idxT.at[pl.ds(j * NP + base, 128)],
                                idxv.at[j])
            for j in range(FD):
                for i in range(8):
                    dsi = pl.ds(i * 16, 16)
                    idxv[j, dsi] = idxv[j, dsi] + emb_off
            cps = [pltpu.async_copy(embf.at[idxv.at[j]], gbuf.at[j], sem)
                   for j in range(FD)]
            pltpu.sync_copy(cont.at[pl.ds(y_off + base, 128)], cbuf)
            for cp in cps:
                cp.wait()

            def vbody(r, _):
                for q in range(2):
                    dsq = pl.ds(q * 16, 16)
                    v = ((gbuf[0, r, dsq] + gbuf[1, r, dsq])
                         + (gbuf[2, r, dsq] + gbuf[3, r, dsq]) + cbuf[r, dsq])
                    ybuf[r, dsq] = jnp.maximum(v, 0.0)
                return 0

            lax.fori_loop(0, 128, vbody, 0)
            pltpu.sync_copy(ybuf, yout.at[pl.ds(y_off + base, 128)])
            return 0

        lax.fori_loop(0, nk, ebody, 0)

    pltpu.sync_copy(ones16, onev)
    pltpu.sync_copy(zeros16, fbuf16)
    fb0 = s * ROWS_PT
    for t in range(4):
        pltpu.sync_copy(fbuf16, cspm_a.at[pl.ds(fb0 + t * FB, FB)])
        pltpu.sync_copy(fbuf16, cspm_b.at[pl.ds(fb0 + t * FB, FB)])
    plsc.subcore_barrier()

    half_base = c * HALF
    trash = HALF + lax.iota(jnp.int32, 16)
    nch = jnp.minimum(ECPT, ECHUNK - s * ECPT)
    nsuper = nch // 8
    ntail = nch % 8

    for dsrc, dloc, cspm in ((du_, dlu, cspm_a), (di_, dli, cspm_b)):
        def pchunk(j, chunk_id, dsrc=dsrc, dloc=dloc, cspm=cspm):
            ebase = chunk_id * 128
            pltpu.sync_copy(dsrc.at[pl.ds(ebase, 128)], dbuf.at[j])
            for i in range(8):
                dsi = pl.ds(i * 16, 16)
                loc = dbuf[j, dsi] - half_base
                ok = (loc >= 0) & (loc < HALF)
                lbuf[j, dsi] = jnp.where(ok, loc, trash)
            pltpu.sync_copy(lbuf.at[j], dloc.at[pl.ds(c * E + ebase, 128)])
            pltpu.sync_copy(onev, cspm.at[lbuf.at[j]], add=True)

        def sbody(k, _, pchunk=pchunk):
            for j in range(8):
                pchunk(j, s * ECPT + k * 8 + j)
            return 0

        lax.fori_loop(0, nsuper, sbody, 0)

        def tbody(k, _, pchunk=pchunk, nsuper=nsuper):
            pchunk(0, s * ECPT + nsuper * 8 + k)
            return 0

        lax.fori_loop(0, ntail, tbody, 0)

    plsc.subcore_barrier()
    for cspm, cout in ((cspm_a, cnti2), (cspm_b, cntu2)):
        for t in range(4):
            pltpu.sync_copy(cspm.at[pl.ds(fb0 + t * FB, FB)], fbuf16)
            pltpu.sync_copy(
                fbuf16, cout.at[pl.ds(half_base + fb0 + t * FB, FB)])


@functools.lru_cache(maxsize=None)
def _sc_prep():
  return pl.kernel(
    _sc_prep_body,
    out_type=(_f32(2 * NP, HD), _f32(2 * NP, HD), _f32(NP, 16), _f32(NP, 16),
              _i32(2 * E), _i32(2 * E)),
    mesh=_mesh(),
    scratch_types=[
        pltpu.VMEM((FD, 128), jnp.int32),
        pltpu.VMEM((FD, 128, HD), jnp.float32),
        pltpu.VMEM((128, HD), jnp.float32),
        pltpu.VMEM((128, HD), jnp.float32),
        pltpu.VMEM((8, 128), jnp.int32),
        pltpu.VMEM((8, 128), jnp.int32),
        pltpu.VMEM((128, 16), jnp.float32),
        pltpu.VMEM((FB, 16), jnp.float32),
        pltpu.SemaphoreType.DMA,
        pltpu.VMEM_SHARED((SCR, 16), jnp.float32),
        pltpu.VMEM_SHARED((SCR, 16), jnp.float32),
    ],
    compiler_params=_SC_PARAMS,
  )


@functools.lru_cache(maxsize=None)
def _make_sc_agg(n_tasks):
    def body(*refs):
        ins = refs[:3 * n_tasks]
        zeros32 = refs[3 * n_tasks]
        outs = refs[3 * n_tasks + 1: 4 * n_tasks + 1]
        sbuf, lbuf, rows, zbuf, fbuf, sem, aspm = refs[4 * n_tasks + 1:]
        c = lax.axis_index("c")
        s = lax.axis_index("s")
        y_off = c * NP
        fb0 = s * ROWS_PT
        pltpu.sync_copy(zeros32, zbuf)
        nch = jnp.minimum(ECPT, ECHUNK - s * ECPT)
        nsuper = nch // 8
        ntail = nch % 8
        for ti in range(n_tasks):
            ysrc, src, dloc = ins[3 * ti: 3 * ti + 3]
            agg = outs[ti]
            for p in range(2):
                for t in range(4):
                    pltpu.sync_copy(zbuf, aspm.at[pl.ds(fb0 + t * FB, FB)])
                plsc.subcore_barrier()

                def pchunk(j, chunk_id, ysrc=ysrc, src=src, dloc=dloc, p=p):
                    ebase = chunk_id * 128
                    pltpu.sync_copy(src.at[pl.ds(ebase, 128)], sbuf.at[j])
                    for i in range(8):
                        dsi = pl.ds(i * 16, 16)
                        sbuf[j, dsi] = sbuf[j, dsi] + y_off
                    pltpu.sync_copy(dloc.at[pl.ds(p * E + ebase, 128)],
                                    lbuf.at[j])
                    pltpu.async_copy(
                        ysrc.at[sbuf.at[j]], rows.at[j], sem).wait()
                    pltpu.sync_copy(rows.at[j], aspm.at[lbuf.at[j]],
                                    add=True)

                def sbody(k, _, pchunk=pchunk):
                    for j in range(8):
                        pchunk(j, s * ECPT + k * 8 + j)
                    return 0

                lax.fori_loop(0, nsuper, sbody, 0)

                def tbody(k, _, pchunk=pchunk, nsuper=nsuper):
                    pchunk(0, s * ECPT + nsuper * 8 + k)
                    return 0

                lax.fori_loop(0, ntail, tbody, 0)
                plsc.subcore_barrier()
                out_base = y_off + p * HALF + fb0
                for t in range(4):
                    pltpu.sync_copy(aspm.at[pl.ds(fb0 + t * FB, FB)], fbuf)
                    pltpu.sync_copy(fbuf, agg.at[pl.ds(out_base + t * FB,
                                                       FB)])

    return pl.kernel(
        body,
        out_type=tuple(_f32(2 * NP, HD) for _ in range(n_tasks)),
        mesh=_mesh(),
        scratch_types=[
            pltpu.VMEM((8, 128), jnp.int32),
            pltpu.VMEM((8, 128), jnp.int32),
            pltpu.VMEM((8, 128, HD), jnp.float32),
            pltpu.VMEM((FB, HD), jnp.float32),
            pltpu.VMEM((FB, HD), jnp.float32),
            pltpu.SemaphoreType.DMA,
            pltpu.VMEM_SHARED((SCR, HD), jnp.float32),
        ],
        compiler_params=_SC_PARAMS,
    )


def _dot(a, b):
    return jnp.dot(a, b, preferred_element_type=jnp.float32)


def _row_spec(width):
    return pl.BlockSpec((BLK, width), lambda i, h: (i, 0))


def _row_spec_hi(width):
    return pl.BlockSpec((BLK, width), lambda i, h: (NP // BLK + i, 0))


def _stk_spec(width):
    return pl.BlockSpec((BLK, width), lambda i, h: (h * (NP // BLK) + i, 0))


def _w_spec(r, cdim):
    return pl.BlockSpec((1, r, cdim), lambda i, h: (0, 0, 0))


def _wh_spec(r, cdim):
    return pl.BlockSpec((1, r, cdim), lambda i, h: (h, 0, 0))


def _cont_body(xu, wu, bu, xi, wi, bi, cu, ci):
    cu[...] = _dot(xu[...], wu[...][0]) + bu[...][0]
    ci[...] = _dot(xi[...], wi[...][0]) + bi[...][0]


_cont_call = pl.pallas_call(
    _cont_body,
    grid=(NP // BLK, 2),
    in_specs=[_row_spec(FC), _wh_spec(FC, HD), _wh_spec(1, HD)] * 2,
    out_specs=[_stk_spec(HD)] * 2,
    out_shape=[_f32(2 * NP, HD)] * 2,
)


def _sage_half(aggA, aggB, cnt2, yA, yB, wn, wr, b):
    inv = 1.0 / jnp.maximum(cnt2[...][:, 0:1], 1.0)
    agg = jnp.concatenate([aggA[...], aggB[...]], axis=1)
    y = jnp.concatenate([yA[...], yB[...]], axis=1)
    return _dot(agg * inv, wn[...][0]) + _dot(y, wr[...][0]) + b[...][0]


def _xform_body(aggiA, aggiB, cnti, yiA, yiB, wni, wri, bi_,
                agguA, agguB, cntu, yuA, yuB, wnu, wru, bu_, oi, ou):
    oi[...] = _sage_half(aggiA, aggiB, cnti, yiA, yiB, wni, wri, bi_)
    ou[...] = _sage_half(agguA, agguB, cntu, yuA, yuB, wnu, wru, bu_)


_xform_call = pl.pallas_call(
    _xform_body,
    grid=(NP // BLK, 2),
    in_specs=[_row_spec(HD), _row_spec_hi(HD), _row_spec(16),
              _row_spec(HD), _row_spec_hi(HD),
              _wh_spec(D, HD), _wh_spec(D, HD), _wh_spec(1, HD)] * 2,
    out_specs=[_stk_spec(HD)] * 2,
    out_shape=[_f32(2 * NP, HD)] * 2,
)


def _final_body(aggA, aggB, cntu, yA, yB, wnu, wru, bu_, wo, bo, out):
    t = _sage_half(aggA, aggB, cntu, yA, yB, wnu, wru, bu_)
    out[...] = _dot(t, wo[...][0]) + bo[...][0]


_final_call = pl.pallas_call(
    _final_body,
    grid=(NP // BLK, 1),
    in_specs=[_row_spec(HD), _row_spec_hi(HD), _row_spec(16),
              _row_spec(HD), _row_spec_hi(HD),
              _w_spec(D, D), _w_spec(D, D), _w_spec(1, D),
              _w_spec(D, OUT), _w_spec(1, OUT)],
    out_specs=_row_spec(OUT),
    out_shape=_f32(NP, OUT),
)


def kernel(x_d_user, x_c_user, x_d_item, x_c_item, edge_index_u2i,
           edge_index_i2u, emb_user, emb_item, W_c_user, b_c_user, W_c_item,
           b_c_item, Wn_u2i, Wr_u2i, b_u2i, Wn_i2u, Wr_i2u, b_i2u, W_out,
           b_out):
    f32 = jnp.float32
    offs = (jnp.arange(FD, dtype=jnp.int32) * V)[:, None]
    idxu = jnp.pad(x_d_user.astype(jnp.int32).T + offs,
                   ((0, 0), (0, NP - N))).reshape(-1)
    idxi = jnp.pad(x_d_item.astype(jnp.int32).T + offs,
                   ((0, 0), (0, NP - N))).reshape(-1)
    embu = emb_user.reshape(FD * V, D)
    embu = jnp.concatenate([embu[:, :HD], embu[:, HD:]], axis=0)
    embi = emb_item.reshape(FD * V, D)
    embi = jnp.concatenate([embi[:, :HD], embi[:, HD:]], axis=0)
    xcu = jnp.pad(x_c_user.astype(f32), ((0, NP - N), (0, 0)))
    xci = jnp.pad(x_c_item.astype(f32), ((0, NP - N), (0, 0)))
    su, du = edge_index_u2i[0], edge_index_u2i[1]
    si, di = edge_index_i2u[0], edge_index_i2u[1]
    zeros16 = jnp.zeros((FB, 16), f32)
    ones16 = jnp.ones((128, 16), f32)
    zeros32 = jnp.zeros((FB, HD), f32)

    def stk_w(w):
        return jnp.stack([w[:, :HD], w[:, HD:]])

    def stk_b(b):
        return jnp.stack([b[:HD].reshape(1, HD), b[HD:].reshape(1, HD)])

    cont_u, cont_i = _cont_call(
        xcu, stk_w(W_c_user), stk_b(b_c_user),
        xci, stk_w(W_c_item), stk_b(b_c_item))
    yu, yi, cnti2, cntu2, dlu, dli = _sc_prep()(
        idxu, embu, cont_u, idxi, embi, cont_i, du, di, zeros16, ones16)

    agg_i0, agg_u0 = _make_sc_agg(2)(yu, su, dlu, yi, si, dli, zeros32)
    y1i, y1u = _xform_call(
        agg_i0, agg_i0, cnti2, yi, yi,
        stk_w(Wn_u2i[0]), stk_w(Wr_u2i[0]), stk_b(b_u2i[0]),
        agg_u0, agg_u0, cntu2, yu, yu,
        stk_w(Wn_i2u[0]), stk_w(Wr_i2u[0]), stk_b(b_i2u[0]))

    agg_u1, = _make_sc_agg(1)(y1i, si, dli, zeros32)
    out = _final_call(
        agg_u1, agg_u1, cntu2, y1u, y1u,
        Wn_i2u[1][None], Wr_i2u[1][None], b_i2u[1].reshape(1, 1, D),
        W_out[None], b_out.reshape(1, 1, OUT))
    return out[:N]


# speedup vs baseline: 4.7370x; 2.2205x over previous
import functools

import jax
import jax.numpy as jnp
from jax import lax
from jax.experimental import pallas as pl
from jax.experimental.pallas import tpu as pltpu
from jax.experimental.pallas import tpu_sc as plsc

N = 50000
E = 400000
FD = 4
V = 1000
FC = 16
D = 64
HD = D // 2
OUT = 32

NP = 50176
HALF = NP // 2
SCR = HALF + 32
NCHUNK = NP // 128
ECHUNK = E // 128
ECPT = 196
ROWS_PT = HALF // 16
FB = ROWS_PT // 4

_mesh = functools.partial(
    plsc.VectorSubcoreMesh, core_axis_name="c", subcore_axis_name="s",
    num_cores=2, num_subcores=16)

_SC_PARAMS = pltpu.CompilerParams(use_tc_tiling_on_sc=False)

BLK = 1792


def _f32(*shape):
    return jax.ShapeDtypeStruct(shape, jnp.float32)


def _i32(*shape):
    return jax.ShapeDtypeStruct(shape, jnp.int32)


def _sc_prep_body(idxu, embu, contu, idxi, embi, conti, du_, di_,
                  zeros16, ones16,
                  yu, yi, cnti2, cntu2, dlu, dli,
                  idxv, gbuf, cbuf, ybuf, dbuf, lbuf, onev, fbuf16,
                  sem, osem, cspm_a, cspm_b):
    c = lax.axis_index("c")
    s = lax.axis_index("s")

    y_off = c * NP
    for idxT, embf, cont, yout in ((idxu, embu, contu, yu),
                                   (idxi, embi, conti, yi)):
        nk = (NCHUNK - s + 15) // 16

        def ebody(k, _, idxT=idxT, embf=embf, cont=cont, yout=yout):
            cid = s + 16 * k
            base = cid * 128
            pltpu.sync_copy(idxT.at[c * NCHUNK + cid], idxv)
            cps = [pltpu.async_copy(embf.at[idxv.at[j]], gbuf.at[j], sem)
                   for j in range(FD)]
            cps.append(
                pltpu.async_copy(cont.at[pl.ds(y_off + base, 128)], cbuf,
                                 sem))
            for cp in cps:
                cp.wait()

            def vbody(r, _):
                for q in range(2):
                    dsq = pl.ds(q * 16, 16)
                    v = ((gbuf[0, r, dsq] + gbuf[1, r, dsq])
                         + (gbuf[2, r, dsq] + gbuf[3, r, dsq]) + cbuf[r, dsq])
                    ybuf[r, dsq] = jnp.maximum(v, 0.0)
                return 0

            lax.fori_loop(0, 128, vbody, 0)
            pltpu.sync_copy(ybuf, yout.at[pl.ds(y_off + base, 128)])
            return 0

        lax.fori_loop(0, nk, ebody, 0)

    pltpu.sync_copy(ones16, onev)
    pltpu.sync_copy(zeros16, fbuf16)
    fb0 = s * ROWS_PT
    for t in range(4):
        pltpu.sync_copy(fbuf16, cspm_a.at[pl.ds(fb0 + t * FB, FB)])
        pltpu.sync_copy(fbuf16, cspm_b.at[pl.ds(fb0 + t * FB, FB)])
    plsc.subcore_barrier()

    half_base = c * HALF
    trash = HALF + lax.iota(jnp.int32, 16)
    nch = jnp.minimum(ECPT, ECHUNK - s * ECPT)
    nsuper = nch // 8
    ntail = nch % 8

    for dsrc, dloc, cspm in ((du_, dlu, cspm_a), (di_, dli, cspm_b)):
        def pgroup(ch0, nj, dsrc=dsrc, dloc=dloc, cspm=cspm):
            pltpu.sync_copy(dsrc.at[pl.ds(ch0, nj)], dbuf.at[pl.ds(0, nj)])
            for j in range(nj):
                for i in range(8):
                    dsi = pl.ds(i * 16, 16)
                    loc = dbuf[j, dsi] - half_base
                    ok = (loc >= 0) & (loc < HALF)
                    lbuf[j, dsi] = jnp.where(ok, loc, trash)
            pltpu.sync_copy(lbuf.at[pl.ds(0, nj)],
                            dloc.at[pl.ds(c * ECHUNK + ch0, nj)])
            scs = [pltpu.async_copy(onev, cspm.at[lbuf.at[j]], osem,
                                    add=True) for j in range(nj)]
            for d in scs:
                d.wait()

        def sbody(k, _, pgroup=pgroup):
            pgroup(s * ECPT + k * 8, 8)
            return 0

        lax.fori_loop(0, nsuper, sbody, 0)

        def tbody(k, _, pgroup=pgroup, nsuper=nsuper):
            pgroup(s * ECPT + nsuper * 8 + k, 1)
            return 0

        lax.fori_loop(0, ntail, tbody, 0)

    plsc.subcore_barrier()
    for cspm, cout in ((cspm_a, cnti2), (cspm_b, cntu2)):
        for t in range(4):
            pltpu.sync_copy(cspm.at[pl.ds(fb0 + t * FB, FB)], fbuf16)
            pltpu.sync_copy(
                fbuf16, cout.at[pl.ds(half_base + fb0 + t * FB, FB)])


@functools.lru_cache(maxsize=None)
def _sc_prep():
  return pl.kernel(
    _sc_prep_body,
    out_type=(_f32(2 * NP, HD), _f32(2 * NP, HD), _f32(NP, 16), _f32(NP, 16),
              _i32(2 * ECHUNK, 128), _i32(2 * ECHUNK, 128)),
    mesh=_mesh(),
    scratch_types=[
        pltpu.VMEM((FD, 128), jnp.int32),
        pltpu.VMEM((FD, 128, HD), jnp.float32),
        pltpu.VMEM((128, HD), jnp.float32),
        pltpu.VMEM((128, HD), jnp.float32),
        pltpu.VMEM((8, 128), jnp.int32),
        pltpu.VMEM((8, 128), jnp.int32),
        pltpu.VMEM((128, 16), jnp.float32),
        pltpu.VMEM((FB, 16), jnp.float32),
        pltpu.SemaphoreType.DMA,
        pltpu.SemaphoreType.DMA,
        pltpu.VMEM_SHARED((SCR, 16), jnp.float32),
        pltpu.VMEM_SHARED((SCR, 16), jnp.float32),
    ],
    compiler_params=_SC_PARAMS,
  )


@functools.lru_cache(maxsize=None)
def _make_sc_agg(n_tasks):
    def body(*refs):
        ins = refs[:3 * n_tasks]
        zeros32 = refs[3 * n_tasks]
        outs = refs[3 * n_tasks + 1: 4 * n_tasks + 1]
        (sbuf, lbuf, rows, zbuf, fbuf, gsem, ssem,
         aspm) = refs[4 * n_tasks + 1:]
        c = lax.axis_index("c")
        s = lax.axis_index("s")
        fb0 = s * ROWS_PT
        pltpu.sync_copy(zeros32, zbuf)
        nch = jnp.minimum(ECPT, ECHUNK - s * ECPT)
        nsuper = nch // 8
        ntail = nch % 8
        for ti in range(n_tasks):
            ysrc, src, dloc = ins[3 * ti: 3 * ti + 3]
            agg = outs[ti]
            for p in range(2):
                for t in range(4):
                    pltpu.sync_copy(zbuf, aspm.at[pl.ds(fb0 + t * FB, FB)])
                plsc.subcore_barrier()

                def pgroup(ch0, nj, ysrc=ysrc, src=src, dloc=dloc, p=p):
                    pltpu.sync_copy(src.at[pl.ds(c * ECHUNK + ch0, nj)],
                                    sbuf.at[pl.ds(0, nj)])
                    pltpu.sync_copy(dloc.at[pl.ds(p * ECHUNK + ch0, nj)],
                                    lbuf.at[pl.ds(0, nj)])
                    gs = [pltpu.async_copy(ysrc.at[sbuf.at[j]], rows.at[j],
                                           gsem) for j in range(nj)]
                    scs = []
                    for j in range(nj):
                        gs[j].wait()
                        scs.append(
                            pltpu.async_copy(rows.at[j],
                                             aspm.at[lbuf.at[j]], ssem,
                                             add=True))
                    for d in scs:
                        d.wait()

                def sbody(k, _, pgroup=pgroup):
                    pgroup(s * ECPT + k * 8, 8)
                    return 0

                lax.fori_loop(0, nsuper, sbody, 0)

                def tbody(k, _, pgroup=pgroup, nsuper=nsuper):
                    pgroup(s * ECPT + nsuper * 8 + k, 1)
                    return 0

                lax.fori_loop(0, ntail, tbody, 0)
                plsc.subcore_barrier()
                out_base = c * NP + p * HALF + fb0
                for t in range(4):
                    pltpu.sync_copy(aspm.at[pl.ds(fb0 + t * FB, FB)], fbuf)
                    pltpu.sync_copy(fbuf, agg.at[pl.ds(out_base + t * FB,
                                                       FB)])

    return pl.kernel(
        body,
        out_type=tuple(_f32(2 * NP, HD) for _ in range(n_tasks)),
        mesh=_mesh(),
        scratch_types=[
            pltpu.VMEM((8, 128), jnp.int32),
            pltpu.VMEM((8, 128), jnp.int32),
            pltpu.VMEM((8, 128, HD), jnp.float32),
            pltpu.VMEM((FB, HD), jnp.float32),
            pltpu.VMEM((FB, HD), jnp.float32),
            pltpu.SemaphoreType.DMA,
            pltpu.SemaphoreType.DMA,
            pltpu.VMEM_SHARED((SCR, HD), jnp.float32),
        ],
        compiler_params=_SC_PARAMS,
    )


def _dot(a, b):
    return jnp.dot(a, b, preferred_element_type=jnp.float32)


def _row_spec(width):
    return pl.BlockSpec((BLK, width), lambda i, h: (i, 0))


def _row_spec_hi(width):
    return pl.BlockSpec((BLK, width), lambda i, h: (NP // BLK + i, 0))


def _stk_spec(width):
    return pl.BlockSpec((BLK, width), lambda i, h: (h * (NP // BLK) + i, 0))


def _w_spec(r, cdim):
    return pl.BlockSpec((1, r, cdim), lambda i, h: (0, 0, 0))


def _wh_spec(r, cdim):
    return pl.BlockSpec((1, r, cdim), lambda i, h: (h, 0, 0))


def _cont_body(xu, wu, bu, xi, wi, bi, cu, ci):
    cu[...] = _dot(xu[...], wu[...][0]) + bu[...][0]
    ci[...] = _dot(xi[...], wi[...][0]) + bi[...][0]


_cont_call = pl.pallas_call(
    _cont_body,
    grid=(NP // BLK, 2),
    in_specs=[_row_spec(FC), _wh_spec(FC, HD), _wh_spec(1, HD)] * 2,
    out_specs=[_stk_spec(HD)] * 2,
    out_shape=[_f32(2 * NP, HD)] * 2,
)


def _sage_half(aggA, aggB, cnt2, yA, yB, wn, wr, b):
    inv = 1.0 / jnp.maximum(cnt2[...][:, 0:1], 1.0)
    agg = jnp.concatenate([aggA[...], aggB[...]], axis=1)
    y = jnp.concatenate([yA[...], yB[...]], axis=1)
    return _dot(agg * inv, wn[...][0]) + _dot(y, wr[...][0]) + b[...][0]


def _xform_body(aggiA, aggiB, cnti, yiA, yiB, wni, wri, bi_,
                agguA, agguB, cntu, yuA, yuB, wnu, wru, bu_, oi, ou):
    oi[...] = _sage_half(aggiA, aggiB, cnti, yiA, yiB, wni, wri, bi_)
    ou[...] = _sage_half(agguA, agguB, cntu, yuA, yuB, wnu, wru, bu_)


_xform_call = pl.pallas_call(
    _xform_body,
    grid=(NP // BLK, 2),
    in_specs=[_row_spec(HD), _row_spec_hi(HD), _row_spec(16),
              _row_spec(HD), _row_spec_hi(HD),
              _wh_spec(D, HD), _wh_spec(D, HD), _wh_spec(1, HD)] * 2,
    out_specs=[_stk_spec(HD)] * 2,
    out_shape=[_f32(2 * NP, HD)] * 2,
)


def _final_body(aggA, aggB, cntu, yA, yB, wnu, wru, bu_, wo, bo, out):
    t = _sage_half(aggA, aggB, cntu, yA, yB, wnu, wru, bu_)
    out[...] = _dot(t, wo[...][0]) + bo[...][0]


_final_call = pl.pallas_call(
    _final_body,
    grid=(NP // BLK, 1),
    in_specs=[_row_spec(HD), _row_spec_hi(HD), _row_spec(16),
              _row_spec(HD), _row_spec_hi(HD),
              _w_spec(D, D), _w_spec(D, D), _w_spec(1, D),
              _w_spec(D, OUT), _w_spec(1, OUT)],
    out_specs=_row_spec(OUT),
    out_shape=_f32(NP, OUT),
)


def kernel(x_d_user, x_c_user, x_d_item, x_c_item, edge_index_u2i,
           edge_index_i2u, emb_user, emb_item, W_c_user, b_c_user, W_c_item,
           b_c_item, Wn_u2i, Wr_u2i, b_u2i, Wn_i2u, Wr_i2u, b_i2u, W_out,
           b_out):
    f32 = jnp.float32
    offs = (jnp.arange(FD, dtype=jnp.int32) * V)[:, None]

    def prep_idx(x_d):
        ix = jnp.pad(x_d.astype(jnp.int32).T + offs, ((0, 0), (0, NP - N)))
        ix = ix.reshape(FD, NCHUNK, 128).transpose(1, 0, 2)
        return jnp.concatenate([ix, ix + FD * V], axis=0)

    idxu = prep_idx(x_d_user)
    idxi = prep_idx(x_d_item)
    embu = emb_user.reshape(FD * V, D)
    embu = jnp.concatenate([embu[:, :HD], embu[:, HD:]], axis=0)
    embi = emb_item.reshape(FD * V, D)
    embi = jnp.concatenate([embi[:, :HD], embi[:, HD:]], axis=0)
    xcu = jnp.pad(x_c_user.astype(f32), ((0, NP - N), (0, 0)))
    xci = jnp.pad(x_c_item.astype(f32), ((0, NP - N), (0, 0)))

    def prep_src(srow):
        return jnp.concatenate([srow, srow + NP]).reshape(2 * ECHUNK, 128)

    su2 = prep_src(edge_index_u2i[0])
    si2 = prep_src(edge_index_i2u[0])
    du2 = edge_index_u2i[1].reshape(ECHUNK, 128)
    di2 = edge_index_i2u[1].reshape(ECHUNK, 128)
    zeros16 = jnp.zeros((FB, 16), f32)
    ones16 = jnp.ones((128, 16), f32)
    zeros32 = jnp.zeros((FB, HD), f32)

    def stk_w(w):
        return jnp.stack([w[:, :HD], w[:, HD:]])

    def stk_b(b):
        return jnp.stack([b[:HD].reshape(1, HD), b[HD:].reshape(1, HD)])

    cont_u, cont_i = _cont_call(
        xcu, stk_w(W_c_user), stk_b(b_c_user),
        xci, stk_w(W_c_item), stk_b(b_c_item))
    yu, yi, cnti2, cntu2, dlu, dli = _sc_prep()(
        idxu, embu, cont_u, idxi, embi, cont_i, du2, di2, zeros16, ones16)

    agg_i0, agg_u0 = _make_sc_agg(2)(yu, su2, dlu, yi, si2, dli, zeros32)
    y1i, y1u = _xform_call(
        agg_i0, agg_i0, cnti2, yi, yi,
        stk_w(Wn_u2i[0]), stk_w(Wr_u2i[0]), stk_b(b_u2i[0]),
        agg_u0, agg_u0, cntu2, yu, yu,
        stk_w(Wn_i2u[0]), stk_w(Wr_i2u[0]), stk_b(b_i2u[0]))

    agg_u1, = _make_sc_agg(1)(y1i, si2, dli, zeros32)
    out = _final_call(
        agg_u1, agg_u1, cntu2, y1u, y1u,
        Wn_i2u[1][None], Wr_i2u[1][None], b_i2u[1].reshape(1, 1, D),
        W_out[None], b_out.reshape(1, 1, OUT))
    return out[:N]


# speedup vs baseline: 5.0236x; 1.0605x over previous
import functools

import jax
import jax.numpy as jnp
from jax import lax
from jax.experimental import pallas as pl
from jax.experimental.pallas import tpu as pltpu
from jax.experimental.pallas import tpu_sc as plsc

N = 50000
E = 400000
FD = 4
V = 1000
FC = 16
D = 64
HD = D // 2
OUT = 32

NP = 50176
HALF = NP // 2
SCR = HALF + 32
NCHUNK = NP // 128
ECHUNK = E // 128
ECPT = 196
ROWS_PT = HALF // 16
FB = ROWS_PT // 4
FBC = ROWS_PT // 8

_mesh = functools.partial(
    plsc.VectorSubcoreMesh, core_axis_name="c", subcore_axis_name="s",
    num_cores=2, num_subcores=16)

_SC_PARAMS = pltpu.CompilerParams(use_tc_tiling_on_sc=False)

BLK = 1792


def _f32(*shape):
    return jax.ShapeDtypeStruct(shape, jnp.float32)


def _i32(*shape):
    return jax.ShapeDtypeStruct(shape, jnp.int32)


def _sc_prep_body(idxu, embu, contu, idxi, embi, conti, du_, di_,
                  zeros16, ones16,
                  yu, yi, cnti2, cntu2, dlu, dli,
                  idxv, gbuf, cbuf, ybuf, dbuf, lbuf, onev, fbuf16,
                  sem, osem, cspm_a, cspm_b):
    c = lax.axis_index("c")
    s = lax.axis_index("s")

    y_off = c * NP
    for idxT, embf, cont, yout in ((idxu, embu, contu, yu),
                                   (idxi, embi, conti, yi)):
        nk = (NCHUNK - s + 15) // 16

        def ebody(k, _, idxT=idxT, embf=embf, cont=cont, yout=yout):
            cid = s + 16 * k
            base = cid * 128
            pltpu.sync_copy(idxT.at[c * NCHUNK + cid], idxv)
            cps = [pltpu.async_copy(embf.at[idxv.at[j]], gbuf.at[j], sem)
                   for j in range(FD)]
            cps.append(
                pltpu.async_copy(cont.at[pl.ds(y_off + base, 128)], cbuf,
                                 sem))
            for cp in cps:
                cp.wait()

            def vbody(r, _):
                for q in range(2):
                    dsq = pl.ds(q * 16, 16)
                    v = ((gbuf[0, r, dsq] + gbuf[1, r, dsq])
                         + (gbuf[2, r, dsq] + gbuf[3, r, dsq]) + cbuf[r, dsq])
                    ybuf[r, dsq] = jnp.maximum(v, 0.0)
                return 0

            lax.fori_loop(0, 128, vbody, 0)
            pltpu.sync_copy(ybuf, yout.at[pl.ds(y_off + base, 128)])
            return 0

        lax.fori_loop(0, nk, ebody, 0)

    pltpu.sync_copy(ones16, onev)
    pltpu.sync_copy(zeros16, fbuf16)
    fb0 = s * ROWS_PT
    for t in range(4):
        pltpu.sync_copy(fbuf16, cspm_a.at[pl.ds(fb0 + t * FB, FB)])
        pltpu.sync_copy(fbuf16, cspm_b.at[pl.ds(fb0 + t * FB, FB)])
    plsc.subcore_barrier()

    half_base = c * HALF
    trash = HALF + lax.iota(jnp.int32, 16)
    nch = jnp.minimum(ECPT, ECHUNK - s * ECPT)
    nsuper = nch // 8
    ntail = nch % 8

    for dsrc, dloc, cspm in ((du_, dlu, cspm_a), (di_, dli, cspm_b)):
        def pgroup(ch0, nj, dsrc=dsrc, dloc=dloc, cspm=cspm):
            pltpu.sync_copy(dsrc.at[pl.ds(ch0, nj)], dbuf.at[pl.ds(0, nj)])
            for j in range(nj):
                for i in range(8):
                    dsi = pl.ds(i * 16, 16)
                    loc = dbuf[j, dsi] - half_base
                    ok = (loc >= 0) & (loc < HALF)
                    lbuf[j, dsi] = jnp.where(ok, loc, trash)
            pltpu.sync_copy(lbuf.at[pl.ds(0, nj)],
                            dloc.at[pl.ds(c * ECHUNK + ch0, nj)])
            scs = [pltpu.async_copy(onev, cspm.at[lbuf.at[j]], osem,
                                    add=True) for j in range(nj)]
            for d in scs:
                d.wait()

        def sbody(k, _, pgroup=pgroup):
            pgroup(s * ECPT + k * 8, 8)
            return 0

        lax.fori_loop(0, nsuper, sbody, 0)

        def tbody(k, _, pgroup=pgroup, nsuper=nsuper):
            pgroup(s * ECPT + nsuper * 8 + k, 1)
            return 0

        lax.fori_loop(0, ntail, tbody, 0)

    plsc.subcore_barrier()
    for cspm, cout in ((cspm_a, cnti2), (cspm_b, cntu2)):
        for t in range(4):
            pltpu.sync_copy(cspm.at[pl.ds(fb0 + t * FB, FB)], fbuf16)
            pltpu.sync_copy(
                fbuf16, cout.at[pl.ds(half_base + fb0 + t * FB, FB)])


@functools.lru_cache(maxsize=None)
def _sc_prep():
  return pl.kernel(
    _sc_prep_body,
    out_type=(_f32(2 * NP, HD), _f32(2 * NP, HD), _f32(NP, 16), _f32(NP, 16),
              _i32(2 * ECHUNK, 128), _i32(2 * ECHUNK, 128)),
    mesh=_mesh(),
    scratch_types=[
        pltpu.VMEM((FD, 128), jnp.int32),
        pltpu.VMEM((FD, 128, HD), jnp.float32),
        pltpu.VMEM((128, HD), jnp.float32),
        pltpu.VMEM((128, HD), jnp.float32),
        pltpu.VMEM((8, 128), jnp.int32),
        pltpu.VMEM((8, 128), jnp.int32),
        pltpu.VMEM((128, 16), jnp.float32),
        pltpu.VMEM((FB, 16), jnp.float32),
        pltpu.SemaphoreType.DMA,
        pltpu.SemaphoreType.DMA,
        pltpu.VMEM_SHARED((SCR, 16), jnp.float32),
        pltpu.VMEM_SHARED((SCR, 16), jnp.float32),
    ],
    compiler_params=_SC_PARAMS,
  )


@functools.lru_cache(maxsize=None)
def _make_sc_agg(n_tasks):
    def body(*refs):
        ins = refs[:3 * n_tasks]
        zeros32 = refs[3 * n_tasks]
        outs = refs[3 * n_tasks + 1: 4 * n_tasks + 1]
        (sbuf, lbuf, rows, fbuf, gsem, ssem,
         aspm) = refs[4 * n_tasks + 1:]
        c = lax.axis_index("c")
        s = lax.axis_index("s")
        fb0 = s * ROWS_PT
        nch = jnp.minimum(ECPT, ECHUNK - s * ECPT)
        nsuper = nch // 8
        ntail = nch % 8
        for ti in range(n_tasks):
            ysrc, src, dloc = ins[3 * ti: 3 * ti + 3]
            agg = outs[ti]
            for p in range(2):
                pltpu.sync_copy(zeros32, fbuf)
                for t in range(8):
                    pltpu.sync_copy(fbuf, aspm.at[pl.ds(fb0 + t * FBC, FBC)])
                plsc.subcore_barrier()

                def fire(q, ch0, ysrc=ysrc, src=src, dloc=dloc, p=p):
                    pltpu.sync_copy(src.at[pl.ds(c * ECHUNK + ch0, 8)],
                                    sbuf.at[q])
                    pltpu.sync_copy(dloc.at[pl.ds(p * ECHUNK + ch0, 8)],
                                    lbuf.at[q])
                    for j in range(8):
                        pltpu.async_copy(ysrc.at[sbuf.at[q, j]],
                                         rows.at[q, j], gsem)

                def mid(q, ysrc=ysrc):
                    for j in range(8):
                        pltpu.make_async_copy(ysrc.at[sbuf.at[q, j]],
                                              rows.at[q, j], gsem).wait()
                        pltpu.async_copy(rows.at[q, j],
                                         aspm.at[lbuf.at[q, j]], ssem,
                                         add=True)

                def drain(q):
                    for j in range(8):
                        pltpu.make_async_copy(rows.at[q, j],
                                              aspm.at[lbuf.at[q, j]],
                                              ssem).wait()

                @pl.when(nsuper > 0)
                def _():
                    fire(0, s * ECPT)

                def sbody(k, _, fire=fire, mid=mid, drain=drain,
                          nsuper=nsuper):
                    q = k % 2

                    @pl.when(k >= 1)
                    def _():
                        drain(1 - q)

                    @pl.when(k + 1 < nsuper)
                    def _():
                        fire(1 - q, s * ECPT + (k + 1) * 8)

                    mid(q)
                    return 0

                lax.fori_loop(0, nsuper, sbody, 0)

                @pl.when(nsuper > 0)
                def _():
                    drain((nsuper - 1) % 2)

                def tbody(k, _, ysrc=ysrc, src=src, dloc=dloc, p=p,
                          nsuper=nsuper):
                    ch0 = s * ECPT + nsuper * 8 + k
                    pltpu.sync_copy(src.at[pl.ds(c * ECHUNK + ch0, 1)],
                                    sbuf.at[0, pl.ds(0, 1)])
                    pltpu.sync_copy(dloc.at[pl.ds(p * ECHUNK + ch0, 1)],
                                    lbuf.at[0, pl.ds(0, 1)])
                    pltpu.async_copy(ysrc.at[sbuf.at[0, 0]], rows.at[0, 0],
                                     gsem).wait()
                    pltpu.async_copy(rows.at[0, 0], aspm.at[lbuf.at[0, 0]],
                                     ssem, add=True).wait()
                    return 0

                lax.fori_loop(0, ntail, tbody, 0)
                plsc.subcore_barrier()
                out_base = c * NP + p * HALF + fb0
                for t in range(8):
                    pltpu.sync_copy(aspm.at[pl.ds(fb0 + t * FBC, FBC)], fbuf)
                    pltpu.sync_copy(fbuf, agg.at[pl.ds(out_base + t * FBC,
                                                       FBC)])

    return pl.kernel(
        body,
        out_type=tuple(_f32(2 * NP, HD) for _ in range(n_tasks)),
        mesh=_mesh(),
        scratch_types=[
            pltpu.VMEM((2, 8, 128), jnp.int32),
            pltpu.VMEM((2, 8, 128), jnp.int32),
            pltpu.VMEM((2, 8, 128, HD), jnp.float32),
            pltpu.VMEM((FBC, HD), jnp.float32),
            pltpu.SemaphoreType.DMA,
            pltpu.SemaphoreType.DMA,
            pltpu.VMEM_SHARED((SCR, HD), jnp.float32),
        ],
        compiler_params=_SC_PARAMS,
    )


def _dot(a, b):
    return jnp.dot(a, b, preferred_element_type=jnp.float32)


def _row_spec(width):
    return pl.BlockSpec((BLK, width), lambda i, h: (i, 0))


def _row_spec_hi(width):
    return pl.BlockSpec((BLK, width), lambda i, h: (NP // BLK + i, 0))


def _stk_spec(width):
    return pl.BlockSpec((BLK, width), lambda i, h: (h * (NP // BLK) + i, 0))


def _w_spec(r, cdim):
    return pl.BlockSpec((1, r, cdim), lambda i, h: (0, 0, 0))


def _wh_spec(r, cdim):
    return pl.BlockSpec((1, r, cdim), lambda i, h: (h, 0, 0))


def _cont_body(xu, wu, bu, xi, wi, bi, cu, ci):
    cu[...] = _dot(xu[...], wu[...][0]) + bu[...][0]
    ci[...] = _dot(xi[...], wi[...][0]) + bi[...][0]


_cont_call = pl.pallas_call(
    _cont_body,
    grid=(NP // BLK, 2),
    in_specs=[_row_spec(FC), _wh_spec(FC, HD), _wh_spec(1, HD)] * 2,
    out_specs=[_stk_spec(HD)] * 2,
    out_shape=[_f32(2 * NP, HD)] * 2,
)


def _sage_half(aggA, aggB, cnt2, yA, yB, wn, wr, b):
    inv = 1.0 / jnp.maximum(cnt2[...][:, 0:1], 1.0)
    agg = jnp.concatenate([aggA[...], aggB[...]], axis=1)
    y = jnp.concatenate([yA[...], yB[...]], axis=1)
    return _dot(agg * inv, wn[...][0]) + _dot(y, wr[...][0]) + b[...][0]


def _xform_body(aggiA, aggiB, cnti, yiA, yiB, wni, wri, bi_,
                agguA, agguB, cntu, yuA, yuB, wnu, wru, bu_, oi, ou):
    oi[...] = _sage_half(aggiA, aggiB, cnti, yiA, yiB, wni, wri, bi_)
    ou[...] = _sage_half(agguA, agguB, cntu, yuA, yuB, wnu, wru, bu_)


_xform_call = pl.pallas_call(
    _xform_body,
    grid=(NP // BLK, 2),
    in_specs=[_row_spec(HD), _row_spec_hi(HD), _row_spec(16),
              _row_spec(HD), _row_spec_hi(HD),
              _wh_spec(D, HD), _wh_spec(D, HD), _wh_spec(1, HD)] * 2,
    out_specs=[_stk_spec(HD)] * 2,
    out_shape=[_f32(2 * NP, HD)] * 2,
)


def _final_body(aggA, aggB, cntu, yA, yB, wnu, wru, bu_, wo, bo, out):
    t = _sage_half(aggA, aggB, cntu, yA, yB, wnu, wru, bu_)
    out[...] = _dot(t, wo[...][0]) + bo[...][0]


_final_call = pl.pallas_call(
    _final_body,
    grid=(NP // BLK, 1),
    in_specs=[_row_spec(HD), _row_spec_hi(HD), _row_spec(16),
              _row_spec(HD), _row_spec_hi(HD),
              _w_spec(D, D), _w_spec(D, D), _w_spec(1, D),
              _w_spec(D, OUT), _w_spec(1, OUT)],
    out_specs=_row_spec(OUT),
    out_shape=_f32(NP, OUT),
)


def kernel(x_d_user, x_c_user, x_d_item, x_c_item, edge_index_u2i,
           edge_index_i2u, emb_user, emb_item, W_c_user, b_c_user, W_c_item,
           b_c_item, Wn_u2i, Wr_u2i, b_u2i, Wn_i2u, Wr_i2u, b_i2u, W_out,
           b_out):
    f32 = jnp.float32
    offs = (jnp.arange(FD, dtype=jnp.int32) * V)[:, None]

    def prep_idx(x_d):
        ix = jnp.pad(x_d.astype(jnp.int32).T + offs, ((0, 0), (0, NP - N)))
        ix = ix.reshape(FD, NCHUNK, 128).transpose(1, 0, 2)
        return jnp.concatenate([ix, ix + FD * V], axis=0)

    idxu = prep_idx(x_d_user)
    idxi = prep_idx(x_d_item)
    embu = emb_user.reshape(FD * V, D)
    embu = jnp.concatenate([embu[:, :HD], embu[:, HD:]], axis=0)
    embi = emb_item.reshape(FD * V, D)
    embi = jnp.concatenate([embi[:, :HD], embi[:, HD:]], axis=0)
    xcu = jnp.pad(x_c_user.astype(f32), ((0, NP - N), (0, 0)))
    xci = jnp.pad(x_c_item.astype(f32), ((0, NP - N), (0, 0)))

    def prep_src(srow):
        return jnp.concatenate([srow, srow + NP]).reshape(2 * ECHUNK, 128)

    su2 = prep_src(edge_index_u2i[0])
    si2 = prep_src(edge_index_i2u[0])
    du2 = edge_index_u2i[1].reshape(ECHUNK, 128)
    di2 = edge_index_i2u[1].reshape(ECHUNK, 128)
    zeros16 = jnp.zeros((FB, 16), f32)
    ones16 = jnp.ones((128, 16), f32)
    zeros32 = jnp.zeros((FBC, HD), f32)

    def stk_w(w):
        return jnp.stack([w[:, :HD], w[:, HD:]])

    def stk_b(b):
        return jnp.stack([b[:HD].reshape(1, HD), b[HD:].reshape(1, HD)])

    cont_u, cont_i = _cont_call(
        xcu, stk_w(W_c_user), stk_b(b_c_user),
        xci, stk_w(W_c_item), stk_b(b_c_item))
    yu, yi, cnti2, cntu2, dlu, dli = _sc_prep()(
        idxu, embu, cont_u, idxi, embi, cont_i, du2, di2, zeros16, ones16)

    agg_i0, agg_u0 = _make_sc_agg(2)(yu, su2, dlu, yi, si2, dli, zeros32)
    y1i, y1u = _xform_call(
        agg_i0, agg_i0, cnti2, yi, yi,
        stk_w(Wn_u2i[0]), stk_w(Wr_u2i[0]), stk_b(b_u2i[0]),
        agg_u0, agg_u0, cntu2, yu, yu,
        stk_w(Wn_i2u[0]), stk_w(Wr_i2u[0]), stk_b(b_i2u[0]))

    agg_u1, = _make_sc_agg(1)(y1i, si2, dli, zeros32)
    out = _final_call(
        agg_u1, agg_u1, cntu2, y1u, y1u,
        Wn_i2u[1][None], Wr_i2u[1][None], b_i2u[1].reshape(1, 1, D),
        W_out[None], b_out.reshape(1, 1, OUT))
    return out[:N]


# speedup vs baseline: 5.9937x; 1.1931x over previous
import functools

import jax
import jax.numpy as jnp
from jax import lax
from jax.experimental import pallas as pl
from jax.experimental.pallas import tpu as pltpu
from jax.experimental.pallas import tpu_sc as plsc

N = 50000
E = 400000
FD = 4
V = 1000
FC = 16
D = 64
HD = D // 2
OUT = 32

NP = 50176
NP4 = NP // 4
HALF = NP // 2
SCR = HALF + 32
NCHUNK = NP // 128
ECHUNK = E // 128
ECPT = 196
EG = 6
ROWS_PT = HALF // 16
FB = ROWS_PT // 4
FBC = ROWS_PT // 8

_mesh = functools.partial(
    plsc.VectorSubcoreMesh, core_axis_name="c", subcore_axis_name="s",
    num_cores=2, num_subcores=16)

_SC_PARAMS = pltpu.CompilerParams(use_tc_tiling_on_sc=False,
                                  needs_layout_passes=False)

BLK = 1792
BLK4 = BLK // 4
NB = NP // BLK


def _f32(*shape):
    return jax.ShapeDtypeStruct(shape, jnp.float32)


def _i32(*shape):
    return jax.ShapeDtypeStruct(shape, jnp.int32)


def _sc_prep_body(idxu, embu, contu, idxi, embi, conti, du_, di_,
                  zeros16, ones16,
                  yu, yi, invi4, invu4, dlu, dli,
                  idxv, gbuf, cbuf, ybuf, dbuf, lbuf, onev, fbuf16, ibuf,
                  sem, cspm_a, cspm_b):
    c = lax.axis_index("c")
    s = lax.axis_index("s")

    y_off = c * NP
    for idxT, embf, cont, yout in ((idxu, embu, contu, yu),
                                   (idxi, embi, conti, yi)):
        nk = (NCHUNK - s + 15) // 16

        def ebody(k, _, idxT=idxT, embf=embf, cont=cont, yout=yout):
            cid = s + 16 * k
            base4 = (y_off + cid * 128) // 4
            pltpu.sync_copy(idxT.at[c * NCHUNK + cid], idxv)
            cps = [pltpu.async_copy(embf.at[idxv.at[j]], gbuf.at[j], sem)
                   for j in range(FD)]
            cps.append(
                pltpu.async_copy(cont.at[pl.ds(base4, 32)], cbuf, sem))
            for cp in cps:
                cp.wait()

            def vbody(r, _):
                r4 = r // 4
                co = (r % 4) * HD
                for q in range(2):
                    dsq = pl.ds(q * 16, 16)
                    dsc = pl.ds(co + q * 16, 16)
                    v = ((gbuf[0, r, dsq] + gbuf[1, r, dsq])
                         + (gbuf[2, r, dsq] + gbuf[3, r, dsq])
                         + cbuf[r4, dsc])
                    ybuf[r4, dsc] = jnp.maximum(v, 0.0)
                return 0

            lax.fori_loop(0, 128, vbody, 0)
            pltpu.sync_copy(ybuf, yout.at[pl.ds(base4, 32)])
            return 0

        lax.fori_loop(0, nk, ebody, 0)

    pltpu.sync_copy(ones16, onev)
    pltpu.sync_copy(zeros16, fbuf16)
    fb0 = s * ROWS_PT
    for t in range(4):
        pltpu.sync_copy(fbuf16, cspm_a.at[pl.ds(fb0 + t * FB, FB)])
        pltpu.sync_copy(fbuf16, cspm_b.at[pl.ds(fb0 + t * FB, FB)])
    plsc.subcore_barrier()

    half_base = c * HALF
    trash = HALF + lax.iota(jnp.int32, 16)
    nch = jnp.minimum(ECPT, ECHUNK - s * ECPT)
    nsuper = nch // 8
    ntail = nch % 8

    for dsrc, dloc, cspm in ((du_, dlu, cspm_a), (di_, dli, cspm_b)):
        def pgroup(ch0, nj, dsrc=dsrc, dloc=dloc, cspm=cspm):
            pltpu.sync_copy(dsrc.at[pl.ds(ch0, nj)], dbuf.at[pl.ds(0, nj)])
            for j in range(nj):
                for i in range(8):
                    dsi = pl.ds(i * 16, 16)
                    loc = dbuf[j, dsi] - half_base
                    ok = (loc >= 0) & (loc < HALF)
                    lbuf[j, dsi] = jnp.where(ok, loc, trash)
            pltpu.sync_copy(lbuf.at[pl.ds(0, nj)],
                            dloc.at[pl.ds(c * ECHUNK + ch0, nj)])
            scs = [pltpu.async_copy(onev, cspm.at[lbuf.at[j]], sem,
                                    add=True) for j in range(nj)]
            for d in scs:
                d.wait()

        def sbody(k, _, pgroup=pgroup):
            pgroup(s * ECPT + k * 8, 8)
            return 0

        lax.fori_loop(0, nsuper, sbody, 0)

        def tbody(k, _, pgroup=pgroup, nsuper=nsuper):
            pgroup(s * ECPT + nsuper * 8 + k, 1)
            return 0

        lax.fori_loop(0, ntail, tbody, 0)

    plsc.subcore_barrier()
    zero16i = jnp.zeros((16,), jnp.int32)
    for cspm, iout in ((cspm_a, invi4), (cspm_b, invu4)):
        for t in range(4):
            pltpu.sync_copy(cspm.at[pl.ds(fb0 + t * FB, FB)], fbuf16)

            def rbody(r, _):
                for m in range(8):
                    idxr = jnp.full((16,), 4 * r + m // 2, jnp.int32)
                    cv = plsc.load_gather(fbuf16, [idxr, zero16i])
                    ibuf[r, pl.ds(m * 16, 16)] = (
                        1.0 / jnp.maximum(cv, 1.0))
                return 0

            lax.fori_loop(0, FB // 4, rbody, 0)
            pltpu.sync_copy(
                ibuf, iout.at[pl.ds((half_base + fb0 + t * FB) // 4,
                                    FB // 4)])


@functools.lru_cache(maxsize=None)
def _sc_prep():
  return pl.kernel(
    _sc_prep_body,
    out_type=(_f32(2 * NP4, 128), _f32(2 * NP4, 128),
              _f32(NP4, 128), _f32(NP4, 128),
              _i32(2 * ECHUNK, 128), _i32(2 * ECHUNK, 128)),
    mesh=_mesh(),
    scratch_types=[
        pltpu.VMEM((FD, 128), jnp.int32),
        pltpu.VMEM((FD, 128, HD), jnp.float32),
        pltpu.VMEM((32, 128), jnp.float32),
        pltpu.VMEM((32, 128), jnp.float32),
        pltpu.VMEM((8, 128), jnp.int32),
        pltpu.VMEM((8, 128), jnp.int32),
        pltpu.VMEM((128, 16), jnp.float32),
        pltpu.VMEM((FB, 16), jnp.float32),
        pltpu.VMEM((FB // 4, 128), jnp.float32),
        pltpu.SemaphoreType.DMA,
        pltpu.VMEM_SHARED((SCR, 16), jnp.float32),
        pltpu.VMEM_SHARED((SCR, 16), jnp.float32),
    ],
    compiler_params=_SC_PARAMS,
  )


@functools.lru_cache(maxsize=None)
def _make_sc_agg(n_tasks):
    def body(*refs):
        ins = refs[:3 * n_tasks]
        zeros32 = refs[3 * n_tasks]
        outs = refs[3 * n_tasks + 1: 4 * n_tasks + 1]
        (sbuf, lbuf, rows, fbufa, fbufb, gsem, ssem,
         aspm) = refs[4 * n_tasks + 1:]
        c = lax.axis_index("c")
        s = lax.axis_index("s")
        y_off = c * NP
        fb0 = s * ROWS_PT
        nch = jnp.minimum(ECPT, ECHUNK - s * ECPT)
        nsuper = nch // EG
        ntail = nch % EG
        for ti in range(n_tasks):
            ysrc, src, dloc = ins[3 * ti: 3 * ti + 3]
            agg = outs[ti]
            for p in range(2):
                pltpu.sync_copy(zeros32, fbufa)
                for t in range(8):
                    pltpu.sync_copy(fbufa,
                                    aspm.at[pl.ds(fb0 + t * FBC, FBC)])
                plsc.subcore_barrier()

                def fire(q, ch0, ysrc=ysrc, src=src, dloc=dloc, p=p):
                    pltpu.sync_copy(src.at[pl.ds(ch0, EG)], sbuf.at[q])
                    pltpu.sync_copy(dloc.at[pl.ds(p * ECHUNK + ch0, EG)],
                                    lbuf.at[q])
                    for j in range(EG):
                        for i in range(8):
                            dsi = pl.ds(i * 16, 16)
                            sbuf[q, j, dsi] = sbuf[q, j, dsi] + y_off
                    for j in range(EG):
                        pltpu.async_copy(ysrc.at[sbuf.at[q, j]],
                                         rows.at[q, j], gsem)

                def mid(q, ysrc=ysrc):
                    for j in range(EG):
                        pltpu.make_async_copy(ysrc.at[sbuf.at[q, j]],
                                              rows.at[q, j], gsem).wait()
                        pltpu.async_copy(rows.at[q, j],
                                         aspm.at[lbuf.at[q, j]], ssem,
                                         add=True)

                def drain(q):
                    for j in range(EG):
                        pltpu.make_async_copy(rows.at[q, j],
                                              aspm.at[lbuf.at[q, j]],
                                              ssem).wait()

                @pl.when(nsuper > 0)
                def _():
                    fire(0, s * ECPT)

                def sbody(k, _, fire=fire, mid=mid, drain=drain,
                          nsuper=nsuper):
                    q = k % 2

                    @pl.when(k >= 1)
                    def _():
                        drain(1 - q)

                    @pl.when(k + 1 < nsuper)
                    def _():
                        fire(1 - q, s * ECPT + (k + 1) * EG)

                    mid(q)
                    return 0

                lax.fori_loop(0, nsuper, sbody, 0)

                @pl.when(nsuper > 0)
                def _():
                    drain((nsuper - 1) % 2)

                def tbody(k, _, ysrc=ysrc, src=src, dloc=dloc, p=p,
                          nsuper=nsuper):
                    ch0 = s * ECPT + nsuper * EG + k
                    pltpu.sync_copy(src.at[pl.ds(ch0, 1)],
                                    sbuf.at[0, pl.ds(0, 1)])
                    pltpu.sync_copy(dloc.at[pl.ds(p * ECHUNK + ch0, 1)],
                                    lbuf.at[0, pl.ds(0, 1)])
                    for i in range(8):
                        dsi = pl.ds(i * 16, 16)
                        sbuf[0, 0, dsi] = sbuf[0, 0, dsi] + y_off
                    pltpu.async_copy(ysrc.at[sbuf.at[0, 0]], rows.at[0, 0],
                                     gsem).wait()
                    pltpu.async_copy(rows.at[0, 0], aspm.at[lbuf.at[0, 0]],
                                     ssem, add=True).wait()
                    return 0

                lax.fori_loop(0, ntail, tbody, 0)
                plsc.subcore_barrier()
                out4 = (c * NP + p * HALF + fb0) // 4
                for t in range(8):
                    pltpu.sync_copy(aspm.at[pl.ds(fb0 + t * FBC, FBC)],
                                    fbufa)

                    def rbody(r, _):
                        for m in range(8):
                            fbufb[r, pl.ds(m * 16, 16)] = (
                                fbufa[4 * r + m // 2,
                                      pl.ds((m % 2) * 16, 16)])
                        return 0

                    lax.fori_loop(0, FBC // 4, rbody, 0)
                    pltpu.sync_copy(
                        fbufb, agg.at[pl.ds(out4 + t * (FBC // 4),
                                            FBC // 4)])

    return pl.kernel(
        body,
        out_type=tuple(_f32(2 * NP4, 128) for _ in range(n_tasks)),
        mesh=_mesh(),
        scratch_types=[
            pltpu.VMEM((2, EG, 128), jnp.int32),
            pltpu.VMEM((2, EG, 128), jnp.int32),
            pltpu.VMEM((2, EG, 128, HD), jnp.float32),
            pltpu.VMEM((FBC, HD), jnp.float32),
            pltpu.VMEM((FBC // 4, 128), jnp.float32),
            pltpu.SemaphoreType.DMA,
            pltpu.SemaphoreType.DMA,
            pltpu.VMEM_SHARED((SCR, HD), jnp.float32),
        ],
        compiler_params=_SC_PARAMS,
    )


def _dot(a, b):
    return jnp.dot(a, b, preferred_element_type=jnp.float32)


def _pk_spec():
    return pl.BlockSpec((BLK4, 128), lambda i, h: (i, 0))


def _pk_spec_hi():
    return pl.BlockSpec((BLK4, 128), lambda i, h: (NB + i, 0))


def _pk_stk_spec():
    return pl.BlockSpec((BLK4, 128), lambda i, h: (h * NB + i, 0))


def _w_spec(n):
    return pl.BlockSpec((n, 128, 128), lambda i, h: (0, 0, 0))


def _wh_spec():
    return pl.BlockSpec((1, 128, 128), lambda i, h: (h, 0, 0))


def _bh_spec():
    return pl.BlockSpec((1, 1, 128), lambda i, h: (h, 0, 0))


def _b_spec():
    return pl.BlockSpec((1, 1, 128), lambda i, h: (0, 0, 0))


def _cont_body(xu, wu, bu, xi, wi, bi, cu, ci):
    cu[...] = _dot(xu[...], wu[...][0]) + bu[...][0]
    ci[...] = _dot(xi[...], wi[...][0]) + bi[...][0]


_cont_call = pl.pallas_call(
    _cont_body,
    grid=(NB, 2),
    in_specs=[_pk_spec(), _wh_spec(), _bh_spec()] * 2,
    out_specs=[_pk_stk_spec()] * 2,
    out_shape=[_f32(2 * NP4, 128)] * 2,
)


def _sage_half(aggA, aggB, inv4, yA, yB, wnA, wnB, wrA, wrB, b):
    a = inv4[...]
    return (_dot(aggA[...] * a, wnA) + _dot(aggB[...] * a, wnB)
            + _dot(yA[...], wrA) + _dot(yB[...], wrB) + b)


def _xform_body(aggiA, aggiB, invi, yiA, yiB, wni, wri, bi_,
                agguA, agguB, invu, yuA, yuB, wnu, wru, bu_, oi, ou):
    oi[...] = _sage_half(aggiA, aggiB, invi, yiA, yiB,
                         wni[...][0, 0], wni[...][0, 1], wri[...][0, 0],
                         wri[...][0, 1], bi_[...][0, 0])
    ou[...] = _sage_half(agguA, agguB, invu, yuA, yuB,
                         wnu[...][0, 0], wnu[...][0, 1], wru[...][0, 0],
                         wru[...][0, 1], bu_[...][0, 0])


_xform_call = pl.pallas_call(
    _xform_body,
    grid=(NB, 2),
    in_specs=[_pk_spec(), _pk_spec_hi(), _pk_spec(),
              _pk_spec(), _pk_spec_hi(),
              pl.BlockSpec((1, 2, 128, 128), lambda i, h: (h, 0, 0, 0)),
              pl.BlockSpec((1, 2, 128, 128), lambda i, h: (h, 0, 0, 0)),
              pl.BlockSpec((1, 1, 1, 128), lambda i, h: (h, 0, 0, 0))] * 2,
    out_specs=[_pk_stk_spec()] * 2,
    out_shape=[_f32(2 * NP4, 128)] * 2,
)


def _final_body(aggA, aggB, invu, yA, yB, wn, wr, b, wo, bo, out):
    tA = _sage_half(aggA, aggB, invu, yA, yB,
                    wn[...][0, 0], wn[...][0, 1], wr[...][0, 0],
                    wr[...][0, 1], b[...][0, 0])
    tB = _sage_half(aggA, aggB, invu, yA, yB,
                    wn[...][1, 0], wn[...][1, 1], wr[...][1, 0],
                    wr[...][1, 1], b[...][1, 0])
    out[...] = _dot(tA, wo[...][0]) + _dot(tB, wo[...][1]) + bo[...][0]


_final_call = pl.pallas_call(
    _final_body,
    grid=(NB, 1),
    in_specs=[_pk_spec(), _pk_spec_hi(), _pk_spec(),
              _pk_spec(), _pk_spec_hi(),
              pl.BlockSpec((2, 2, 128, 128), lambda i, h: (0, 0, 0, 0)),
              pl.BlockSpec((2, 2, 128, 128), lambda i, h: (0, 0, 0, 0)),
              pl.BlockSpec((2, 1, 1, 128), lambda i, h: (0, 0, 0, 0)),
              _w_spec(2), _b_spec()],
    out_specs=_pk_spec(),
    out_shape=_f32(NP4, 128),
)


def kernel(x_d_user, x_c_user, x_d_item, x_c_item, edge_index_u2i,
           edge_index_i2u, emb_user, emb_item, W_c_user, b_c_user, W_c_item,
           b_c_item, Wn_u2i, Wr_u2i, b_u2i, Wn_i2u, Wr_i2u, b_i2u, W_out,
           b_out):
    f32 = jnp.float32
    eye4 = jnp.eye(4, dtype=f32)
    offs = (jnp.arange(FD, dtype=jnp.int32) * V)[:, None]

    def prep_idx(x_d):
        ix = jnp.pad(x_d.astype(jnp.int32).T + offs, ((0, 0), (0, NP - N)))
        ix = ix.reshape(FD, NCHUNK, 128).transpose(1, 0, 2)
        return jnp.concatenate([ix, ix + FD * V], axis=0)

    idxu = prep_idx(x_d_user)
    idxi = prep_idx(x_d_item)
    embu = emb_user.reshape(FD * V, D)
    embu = jnp.concatenate([embu[:, :HD], embu[:, HD:]], axis=0)
    embi = emb_item.reshape(FD * V, D)
    embi = jnp.concatenate([embi[:, :HD], embi[:, HD:]], axis=0)

    def prep_xc(x_c):
        xp = jnp.pad(x_c.astype(f32), ((0, NP - N), (0, 0)))
        return jnp.pad(xp.reshape(NP4, 4 * FC), ((0, 0), (0, 128 - 4 * FC)))

    xcu4 = prep_xc(x_c_user)
    xci4 = prep_xc(x_c_item)

    su2 = edge_index_u2i[0].reshape(ECHUNK, 128)
    si2 = edge_index_i2u[0].reshape(ECHUNK, 128)
    du2 = edge_index_u2i[1].reshape(ECHUNK, 128)
    di2 = edge_index_i2u[1].reshape(ECHUNK, 128)
    zeros16 = jnp.zeros((FB, 16), f32)
    ones16 = jnp.ones((128, 16), f32)
    zeros32 = jnp.zeros((FBC, HD), f32)

    def bd4(w32):
        return jnp.kron(eye4, w32.astype(f32))

    def wpair(w):
        return jnp.stack([
            jnp.stack([bd4(w[:HD, h * HD:(h + 1) * HD]),
                       bd4(w[HD:, h * HD:(h + 1) * HD])]) for h in range(2)])

    def bpack(b):
        return jnp.stack([jnp.tile(b[h * HD:(h + 1) * HD], 4).reshape(1, 1,
                                                                      128)
                          for h in range(2)])

    def wcont(w):
        return jnp.stack([
            jnp.pad(jnp.kron(eye4, w[:, h * HD:(h + 1) * HD].astype(f32)),
                    ((0, 128 - 4 * FC), (0, 0))) for h in range(2)])

    cont_u, cont_i = _cont_call(
        xcu4, wcont(W_c_user), bpack(b_c_user)[:, 0],
        xci4, wcont(W_c_item), bpack(b_c_item)[:, 0])
    yu, yi, invi4, invu4, dlu, dli = _sc_prep()(
        idxu, embu, cont_u, idxi, embi, cont_i, du2, di2, zeros16, ones16)

    yu32 = yu.reshape(2 * NP, HD)
    yi32 = yi.reshape(2 * NP, HD)
    agg_i0, agg_u0 = _make_sc_agg(2)(yu32, su2, dlu, yi32, si2, dli, zeros32)
    y1i, y1u = _xform_call(
        agg_i0, agg_i0, invi4, yi, yi,
        wpair(Wn_u2i[0]), wpair(Wr_u2i[0]), bpack(b_u2i[0]),
        agg_u0, agg_u0, invu4, yu, yu,
        wpair(Wn_i2u[0]), wpair(Wr_i2u[0]), bpack(b_i2u[0]))

    agg_u1, = _make_sc_agg(1)(y1i.reshape(2 * NP, HD), si2, dli, zeros32)
    wo4 = jnp.stack([bd4(W_out[:HD]), bd4(W_out[HD:])])
    out = _final_call(
        agg_u1, agg_u1, invu4, y1u, y1u,
        wpair(Wn_i2u[1]), wpair(Wr_i2u[1]), bpack(b_i2u[1]),
        wo4, jnp.tile(b_out, 4).reshape(1, 1, 128))
    return out.reshape(NP, OUT)[:N]
